# baseline (speedup 1.0000x reference)
import functools

import jax
import jax.numpy as jnp
from jax import lax
from jax.experimental import pallas as pl
from jax.experimental.pallas import tpu as pltpu
from jax.experimental.pallas import tpu_sc as plsc

B = 16384
KF = 32
FL = 64


@functools.cache
def _sc_gather():
    info = plsc.get_sparse_core_info()
    nw = info.num_cores * info.num_subcores
    bpw = B // nw
    ch = 128
    nch = bpw // ch
    mesh = plsc.VectorSubcoreMesh(core_axis_name="c", subcore_axis_name="s")

    @functools.partial(
        pl.kernel,
        mesh=mesh,
        compiler_params=pltpu.CompilerParams(use_tc_tiling_on_sc=False),
        out_type=[
            jax.ShapeDtypeStruct((B, KF), jnp.float32),
            jax.ShapeDtypeStruct((B, KF), jnp.float32),
            jax.ShapeDtypeStruct((B, FL), jnp.float32),
        ],
        scratch_types=[
            pltpu.VMEM((bpw,), jnp.int32),
            pltpu.VMEM((bpw,), jnp.int32),
            pltpu.VMEM((bpw, KF), jnp.float32),
            pltpu.VMEM((bpw, KF), jnp.float32),
            pltpu.VMEM((bpw, FL), jnp.float32),
            pltpu.SemaphoreType.DMA,
        ],
    )
    def gather_k(uids, iids, utab, itab, gtab, f1_o, f2_o, f3_o,
                 uidx, iidx, urows, irows, grows, sem):
        wid = lax.axis_index("s") * info.num_cores + lax.axis_index("c")
        base = wid * bpw
        pltpu.sync_copy(uids.at[pl.ds(base, bpw)], uidx)
        pltpu.sync_copy(iids.at[pl.ds(base, bpw)], iidx)
        copies = []
        for j in range(nch):
            sl = pl.ds(j * ch, ch)
            copies.append(pltpu.async_copy(utab.at[uidx.at[sl]], urows.at[sl], sem))
            copies.append(pltpu.async_copy(itab.at[iidx.at[sl]], irows.at[sl], sem))
            copies.append(pltpu.async_copy(gtab.at[iidx.at[sl]], grows.at[sl], sem))
        for c in copies:
            c.wait()
        pltpu.sync_copy(urows, f1_o.at[pl.ds(base, bpw)])
        pltpu.sync_copy(irows, f2_o.at[pl.ds(base, bpw)])
        pltpu.sync_copy(grows, f3_o.at[pl.ds(base, bpw)])

    return gather_k


def _mlp_body(f1_r, f2_r, f3_r, bias_r, Wf_r, bf_r, Wb_r, bb_r,
              W1a_r, W1b_r, W1c_r, b1_r, W2_r, b2_r, W3_r, b3_r, W4_r, b4_r,
              out_r):
    f32 = jnp.float32
    feat = jnp.maximum(
        jnp.dot(f3_r[...], Wf_r[...], preferred_element_type=f32) + bf_r[...], 0.0)
    h = (jnp.dot(f1_r[...], W1a_r[...], preferred_element_type=f32)
         + jnp.dot(f2_r[...], W1b_r[...], preferred_element_type=f32)
         + jnp.dot(feat, W1c_r[...], preferred_element_type=f32)
         + b1_r[...])
    h = jnp.maximum(h, 0.0)
    h = jnp.maximum(jnp.dot(h, W2_r[...], preferred_element_type=f32) + b2_r[...], 0.0)
    h = jnp.maximum(jnp.dot(h, W3_r[...], preferred_element_type=f32) + b3_r[...], 0.0)
    h4 = jnp.dot(h, W4_r[...], preferred_element_type=f32) + b4_r[...]
    out_r[...] = h4 + bias_r[...] * Wb_r[0, 0] + bb_r[...]


def _mlp_call(f1, f2, f3, bias_feat, Wf, bf, Wb, bb,
              W1a, W1b, W1c, b1, W2, b2, W3, b3, W4, b4, *, bm=2048,
              interpret=False):
    grid = (B // bm,)

    def row_spec(d):
        return pl.BlockSpec((bm, d), lambda i: (i, 0))

    def full_spec(a):
        return pl.BlockSpec(a.shape, lambda i: (0,) * a.ndim)

    return pl.pallas_call(
        _mlp_body,
        grid=grid,
        in_specs=[
            row_spec(KF), row_spec(KF), row_spec(FL), row_spec(1),
            full_spec(Wf), full_spec(bf), full_spec(Wb), full_spec(bb),
            full_spec(W1a), full_spec(W1b), full_spec(W1c), full_spec(b1),
            full_spec(W2), full_spec(b2), full_spec(W3), full_spec(b3),
            full_spec(W4), full_spec(b4),
        ],
        out_specs=row_spec(1),
        out_shape=jax.ShapeDtypeStruct((B, 1), jnp.float32),
        interpret=interpret,
    )(f1, f2, f3, bias_feat, Wf, bf, Wb, bb,
      W1a, W1b, W1c, b1, W2, b2, W3, b3, W4, b4)


def kernel(user_ids, item_ids, bias_feat, user_table, item_table, gvec,
           Wf, bf, Wb, bb, W1, b1, W2, b2, W3, b3, W4, b4):
    uids = user_ids.reshape(B).astype(jnp.int32)
    iids = item_ids.reshape(B).astype(jnp.int32)
    f1, f2, f3 = _sc_gather()(uids, iids, user_table, item_table, gvec)
    W1a, W1b, W1c = W1[:KF], W1[KF:2 * KF], W1[2 * KF:]
    return _mlp_call(
        f1, f2, f3, bias_feat, Wf, bf.reshape(1, -1), Wb, bb.reshape(1, 1),
        W1a, W1b, W1c, b1.reshape(1, -1), W2, b2.reshape(1, -1),
        W3, b3.reshape(1, -1), W4, b4.reshape(1, 1))
